# baseline (speedup 1.0000x reference)
import functools

import jax
import jax.numpy as jnp
from jax import lax
from jax.experimental import pallas as pl
from jax.experimental.pallas import tpu as pltpu
from jax.experimental.pallas import tpu_sc as plsc

NUM_WORKERS = 32
LANES = 16
CHUNK = 64
NBUF = 8


@functools.lru_cache(maxsize=None)
def _make_sc_embed(bl, d, l_seq):
    assert bl % (NUM_WORKERS * CHUNK) == 0
    per_w = bl // NUM_WORKERS
    n_chunks = per_w // CHUNK
    assert n_chunks % NBUF == 0 and n_chunks >= 2 * NBUF

    mesh = plsc.VectorSubcoreMesh(core_axis_name="c", subcore_axis_name="s")

    @functools.partial(
        pl.kernel,
        mesh=mesh,
        out_type=jax.ShapeDtypeStruct((bl, d), jnp.float32),
        scratch_types=[
            pltpu.VMEM((per_w,), jnp.int32),
            pltpu.VMEM((per_w,), jnp.int32),
            pltpu.VMEM_SHARED((2 * l_seq, d), jnp.float32),
        ]
        + [pltpu.VMEM((CHUNK, d), jnp.float32) for _ in range(NBUF)]
        + [pltpu.SemaphoreType.DMA for _ in range(3 * NBUF)],
    )
    def sc_embed(x_hbm, seg_hbm, tab_hbm, ext_hbm, out_hbm, xidx, eidx, ext_sh,
                 *bufs_and_sems):
        rows = bufs_and_sems[:NBUF]
        sem_t = bufs_and_sems[NBUF:2 * NBUF]
        sem_e = bufs_and_sems[2 * NBUF:3 * NBUF]
        sem_w = bufs_and_sems[3 * NBUF:4 * NBUF]

        wid = lax.axis_index("s") * 2 + lax.axis_index("c")
        wbase = wid * per_w
        lane = lax.iota(jnp.int32, LANES)

        @pl.when(lax.axis_index("s") == 0)
        def _():
            pltpu.sync_copy(ext_hbm, ext_sh)

        pltpu.sync_copy(x_hbm.at[pl.ds(wbase, per_w)], xidx)
        pltpu.sync_copy(seg_hbm.at[pl.ds(wbase, per_w)], eidx)

        def eidx_body(j, c):
            sl = pl.ds(j * LANES, LANES)
            pos = lax.rem(wbase + j * LANES + lane, jnp.int32(l_seq))
            eidx[sl] = eidx[sl] * jnp.int32(l_seq) + pos
            return c

        def issue_tok(g, p):
            pltpu.async_copy(tab_hbm.at[xidx.at[pl.ds(g * CHUNK, CHUNK)]],
                             rows[p], sem_t[p])

        def issue_ext(g, p):
            pltpu.async_copy(ext_sh.at[eidx.at[pl.ds(g * CHUNK, CHUNK)]],
                             rows[p], sem_e[p], add=True)

        def issue_wb(g, p):
            pltpu.async_copy(rows[p], out_hbm.at[pl.ds(wbase + g * CHUNK, CHUNK)],
                             sem_w[p])

        def wait_gather(sem, p):
            pltpu.make_async_copy(tab_hbm.at[pl.ds(0, CHUNK)], rows[p], sem).wait()

        def wait_ext(sem, p):
            pltpu.make_async_copy(tab_hbm.at[pl.ds(0, CHUNK)], rows[p], sem).wait()

        def wait_wb(p):
            pltpu.make_async_copy(rows[p], out_hbm.at[pl.ds(wbase, CHUNK)],
                                  sem_w[p]).wait()

        for g in range(NBUF):
            issue_tok(g, g)
        lax.fori_loop(0, per_w // LANES, eidx_body, 0)
        plsc.subcore_barrier()
        for g in range(1, NBUF):
            wait_gather(sem_t[g - 1], g - 1)
            issue_ext(g - 1, g - 1)
            if g >= 2:
                wait_ext(sem_e[g - 2], g - 2)
                issue_wb(g - 2, g - 2)

        def block(i, c):
            for p in range(NBUF):
                g = i * NBUF + p
                wait_wb(p)
                issue_tok(g, p)
                pm1 = (p - 1) % NBUF
                wait_gather(sem_t[pm1], pm1)
                issue_ext(g - 1, pm1)
                pm2 = (p - 2) % NBUF
                wait_ext(sem_e[pm2], pm2)
                issue_wb(g - 2, pm2)
            return c

        lax.fori_loop(1, n_chunks // NBUF, block, 0)

        n = n_chunks
        p_last, p_prev = (n - 1) % NBUF, (n - 2) % NBUF
        wait_gather(sem_t[p_last], p_last)
        issue_ext(n - 1, p_last)
        wait_ext(sem_e[p_prev], p_prev)
        issue_wb(n - 2, p_prev)
        wait_ext(sem_e[p_last], p_last)
        issue_wb(n - 1, p_last)
        for p in range(NBUF):
            wait_wb(p)

    return sc_embed


def kernel(x, segment_tokens, token_table, segment_table, pe):
    b, l = x.shape
    d = token_table.shape[1]
    ext = (segment_table[:, None, :] + pe[None, :l, :]).reshape(2 * l, d)
    x_flat = x.reshape(-1).astype(jnp.int32)
    seg_flat = segment_tokens.reshape(-1).astype(jnp.int32)
    out = _make_sc_embed(b * l, d, l)(x_flat, seg_flat, token_table, ext)
    return out.reshape(b, l, d)


# speedup vs baseline: 1.0533x; 1.0533x over previous
import functools

import jax
import jax.numpy as jnp
from jax import lax
from jax.experimental import pallas as pl
from jax.experimental.pallas import tpu as pltpu
from jax.experimental.pallas import tpu_sc as plsc

NUM_WORKERS = 32
LANES = 16
CHUNK = 128
NBUF = 4


@functools.lru_cache(maxsize=None)
def _make_sc_embed(bl, d, l_seq):
    assert bl % (NUM_WORKERS * CHUNK) == 0
    per_w = bl // NUM_WORKERS
    n_chunks = per_w // CHUNK
    assert n_chunks % NBUF == 0 and n_chunks >= 2 * NBUF

    mesh = plsc.VectorSubcoreMesh(core_axis_name="c", subcore_axis_name="s")

    @functools.partial(
        pl.kernel,
        mesh=mesh,
        out_type=jax.ShapeDtypeStruct((bl, d), jnp.float32),
        scratch_types=[
            pltpu.VMEM((per_w,), jnp.int32),
            pltpu.VMEM((per_w,), jnp.int32),
            pltpu.VMEM_SHARED((2 * l_seq, d), jnp.float32),
        ]
        + [pltpu.VMEM((CHUNK, d), jnp.float32) for _ in range(NBUF)]
        + [pltpu.SemaphoreType.DMA for _ in range(3 * NBUF)],
    )
    def sc_embed(x_hbm, seg_hbm, tab_hbm, ext_hbm, out_hbm, xidx, eidx, ext_sh,
                 *bufs_and_sems):
        rows = bufs_and_sems[:NBUF]
        sem_t = bufs_and_sems[NBUF:2 * NBUF]
        sem_e = bufs_and_sems[2 * NBUF:3 * NBUF]
        sem_w = bufs_and_sems[3 * NBUF:4 * NBUF]

        wid = lax.axis_index("s") * 2 + lax.axis_index("c")
        wbase = wid * per_w
        lane = lax.iota(jnp.int32, LANES)

        @pl.when(lax.axis_index("s") == 0)
        def _():
            pltpu.sync_copy(ext_hbm, ext_sh)

        pltpu.sync_copy(x_hbm.at[pl.ds(wbase, per_w)], xidx)
        pltpu.sync_copy(seg_hbm.at[pl.ds(wbase, per_w)], eidx)

        def eidx_body(j, c):
            sl = pl.ds(j * LANES, LANES)
            pos = lax.rem(wbase + j * LANES + lane, jnp.int32(l_seq))
            eidx[sl] = eidx[sl] * jnp.int32(l_seq) + pos
            return c

        def issue_tok(g, p):
            pltpu.async_copy(tab_hbm.at[xidx.at[pl.ds(g * CHUNK, CHUNK)]],
                             rows[p], sem_t[p])

        def issue_ext(g, p):
            pltpu.async_copy(ext_sh.at[eidx.at[pl.ds(g * CHUNK, CHUNK)]],
                             rows[p], sem_e[p], add=True)

        def issue_wb(g, p):
            pltpu.async_copy(rows[p], out_hbm.at[pl.ds(wbase + g * CHUNK, CHUNK)],
                             sem_w[p])

        def wait_gather(sem, p):
            pltpu.make_async_copy(tab_hbm.at[pl.ds(0, CHUNK)], rows[p], sem).wait()

        def wait_ext(sem, p):
            pltpu.make_async_copy(tab_hbm.at[pl.ds(0, CHUNK)], rows[p], sem).wait()

        def wait_wb(p):
            pltpu.make_async_copy(rows[p], out_hbm.at[pl.ds(wbase, CHUNK)],
                                  sem_w[p]).wait()

        for g in range(NBUF):
            issue_tok(g, g)
        lax.fori_loop(0, per_w // LANES, eidx_body, 0)
        plsc.subcore_barrier()
        for g in range(1, NBUF):
            wait_gather(sem_t[g - 1], g - 1)
            issue_ext(g - 1, g - 1)
            if g >= 2:
                wait_ext(sem_e[g - 2], g - 2)
                issue_wb(g - 2, g - 2)

        def block(i, c):
            for p in range(NBUF):
                g = i * NBUF + p
                wait_wb(p)
                issue_tok(g, p)
                pm1 = (p - 1) % NBUF
                wait_gather(sem_t[pm1], pm1)
                issue_ext(g - 1, pm1)
                pm2 = (p - 2) % NBUF
                wait_ext(sem_e[pm2], pm2)
                issue_wb(g - 2, pm2)
            return c

        lax.fori_loop(1, n_chunks // NBUF, block, 0)

        n = n_chunks
        p_last, p_prev = (n - 1) % NBUF, (n - 2) % NBUF
        wait_gather(sem_t[p_last], p_last)
        issue_ext(n - 1, p_last)
        wait_ext(sem_e[p_prev], p_prev)
        issue_wb(n - 2, p_prev)
        wait_ext(sem_e[p_last], p_last)
        issue_wb(n - 1, p_last)
        for p in range(NBUF):
            wait_wb(p)

    return sc_embed


def kernel(x, segment_tokens, token_table, segment_table, pe):
    b, l = x.shape
    d = token_table.shape[1]
    ext = (segment_table[:, None, :] + pe[None, :l, :]).reshape(2 * l, d)
    x_flat = x.reshape(-1).astype(jnp.int32)
    seg_flat = segment_tokens.reshape(-1).astype(jnp.int32)
    out = _make_sc_embed(b * l, d, l)(x_flat, seg_flat, token_table, ext)
    return out.reshape(b, l, d)


# speedup vs baseline: 1.0539x; 1.0006x over previous
import functools

import jax
import jax.numpy as jnp
from jax import lax
from jax.experimental import pallas as pl
from jax.experimental.pallas import tpu as pltpu
from jax.experimental.pallas import tpu_sc as plsc

NUM_WORKERS = 32
LANES = 16
CHUNK = 128
NBUF = 4


@functools.lru_cache(maxsize=None)
def _make_sc_embed(bl, d, l_seq):
    assert bl % (NUM_WORKERS * CHUNK) == 0
    per_w = bl // NUM_WORKERS
    n_chunks = per_w // CHUNK
    assert n_chunks % NBUF == 0 and n_chunks >= 2 * NBUF

    mesh = plsc.VectorSubcoreMesh(core_axis_name="c", subcore_axis_name="s")

    @functools.partial(
        pl.kernel,
        mesh=mesh,
        out_type=jax.ShapeDtypeStruct((bl, d), jnp.float32),
        scratch_types=[
            pltpu.VMEM((per_w,), jnp.int32),
            pltpu.VMEM((per_w,), jnp.int32),
            pltpu.VMEM_SHARED((2 * l_seq, d), jnp.float32),
        ]
        + [pltpu.VMEM((CHUNK, d), jnp.float32) for _ in range(NBUF)]
        + [pltpu.SemaphoreType.DMA for _ in range(3 * NBUF)],
    )
    def sc_embed(x_hbm, seg_hbm, tab_hbm, ext_hbm, out_hbm, xidx, eidx, ext_sh,
                 *bufs_and_sems):
        rows = bufs_and_sems[:NBUF]
        sem_t = bufs_and_sems[NBUF:2 * NBUF]
        sem_e = bufs_and_sems[2 * NBUF:3 * NBUF]
        sem_w = bufs_and_sems[3 * NBUF:4 * NBUF]

        wid = lax.axis_index("s") * 2 + lax.axis_index("c")
        wbase = wid * per_w
        lane = lax.iota(jnp.int32, LANES)

        @pl.when(lax.axis_index("s") == 0)
        def _():
            pltpu.sync_copy(ext_hbm, ext_sh)

        pltpu.sync_copy(x_hbm.at[pl.ds(wbase, per_w)], xidx)
        pltpu.sync_copy(seg_hbm.at[pl.ds(wbase, per_w)], eidx)

        def eidx_body(j, c):
            sl = pl.ds(j * LANES, LANES)
            pos = lax.rem(wbase + j * LANES + lane, jnp.int32(l_seq))
            eidx[sl] = eidx[sl] * jnp.int32(l_seq) + pos
            return c

        def issue_tok(g, p):
            pltpu.async_copy(tab_hbm.at[xidx.at[pl.ds(g * CHUNK, CHUNK)]],
                             rows[p], sem_t[p])

        def issue_ext(g, p):
            pltpu.async_copy(ext_sh.at[eidx.at[pl.ds(g * CHUNK, CHUNK)]],
                             rows[p], sem_e[p], add=True)

        def issue_wb(g, p):
            pltpu.async_copy(rows[p], out_hbm.at[pl.ds(wbase + g * CHUNK, CHUNK)],
                             sem_w[p])

        def wait_gather(sem, p):
            pltpu.make_async_copy(tab_hbm.at[pl.ds(0, CHUNK)], rows[p], sem).wait()

        def wait_ext(sem, p):
            pltpu.make_async_copy(tab_hbm.at[pl.ds(0, CHUNK)], rows[p], sem).wait()

        def wait_wb(p):
            pltpu.make_async_copy(rows[p], out_hbm.at[pl.ds(wbase, CHUNK)],
                                  sem_w[p]).wait()

        for g in range(NBUF):
            issue_tok(g, g)
        lax.fori_loop(0, per_w // LANES, eidx_body, 0)
        plsc.subcore_barrier()
        for g in range(1, NBUF):
            wait_gather(sem_t[g - 1], g - 1)
            issue_ext(g - 1, g - 1)
            if g >= 2:
                wait_ext(sem_e[g - 2], g - 2)
                issue_wb(g - 2, g - 2)

        def block(i, c):
            for p in range(NBUF):
                g = i * NBUF + p
                wait_wb(p)
                issue_tok(g, p)
                pm2 = (p - 2) % NBUF
                wait_ext(sem_e[pm2], pm2)
                issue_wb(g - 2, pm2)
                pm1 = (p - 1) % NBUF
                wait_gather(sem_t[pm1], pm1)
                issue_ext(g - 1, pm1)
            return c

        lax.fori_loop(1, n_chunks // NBUF, block, 0)

        n = n_chunks
        p_last, p_prev = (n - 1) % NBUF, (n - 2) % NBUF
        wait_gather(sem_t[p_last], p_last)
        issue_ext(n - 1, p_last)
        wait_ext(sem_e[p_prev], p_prev)
        issue_wb(n - 2, p_prev)
        wait_ext(sem_e[p_last], p_last)
        issue_wb(n - 1, p_last)
        for p in range(NBUF):
            wait_wb(p)

    return sc_embed


def kernel(x, segment_tokens, token_table, segment_table, pe):
    b, l = x.shape
    d = token_table.shape[1]
    ext = (segment_table[:, None, :] + pe[None, :l, :]).reshape(2 * l, d)
    x_flat = x.reshape(-1).astype(jnp.int32)
    seg_flat = segment_tokens.reshape(-1).astype(jnp.int32)
    out = _make_sc_embed(b * l, d, l)(x_flat, seg_flat, token_table, ext)
    return out.reshape(b, l, d)
